# baseline (speedup 1.0000x reference)
import functools

import jax
import jax.numpy as jnp
from jax import lax
from jax.experimental import pallas as pl
from jax.experimental.pallas import tpu as pltpu
from jax.experimental.pallas import tpu_sc as plsc

_VOCAB = 1000000
_D = 64
_B = 1024
_S = 200
_NC = 2
_NS = 16
_NW = _NC * _NS
_ROWS_PER_W = _B // _NW
_CHUNK = 100
_NCHUNK = _S // _CHUNK
_LANES = 16


def _emb_body(text_hbm, table_hbm, pos_hbm, out_hbm, idx_v, rows_v, pos_v, sem):
    wid = lax.axis_index("s") * _NC + lax.axis_index("c")

    pltpu.sync_copy(pos_hbm, pos_v)

    def row_body(t, carry):
        b = wid * _ROWS_PER_W + t
        pltpu.sync_copy(text_hbm.at[b], idx_v)
        copies = [
            pltpu.async_copy(
                table_hbm.at[idx_v.at[c]],
                rows_v.at[pl.ds(c * _CHUNK, _CHUNK)],
                sem,
            )
            for c in range(_NCHUNK)
        ]
        for cp in copies:
            cp.wait()

        def add_row(r, carry2):
            for j in range(_D // _LANES):
                sl = (r, pl.ds(j * _LANES, _LANES))
                rows_v[sl] = rows_v[sl] + pos_v[sl]
            return carry2

        lax.fori_loop(0, _S, add_row, 0, unroll=2)
        pltpu.sync_copy(rows_v, out_hbm.at[b])
        return carry

    lax.fori_loop(0, _ROWS_PER_W, row_body, 0)


@jax.jit
def _emb(text3, table, pos):
    mesh = plsc.VectorSubcoreMesh(core_axis_name="c", subcore_axis_name="s")
    f = functools.partial(
        pl.kernel,
        mesh=mesh,
        out_type=jax.ShapeDtypeStruct((_B, _S, _D), jnp.float32),
        scratch_types=[
            pltpu.VMEM((_NCHUNK, _CHUNK), jnp.int32),
            pltpu.VMEM((_S, _D), jnp.float32),
            pltpu.VMEM((_S, _D), jnp.float32),
            pltpu.SemaphoreType.DMA,
        ],
        compiler_params=pltpu.CompilerParams(use_tc_tiling_on_sc=False),
    )(_emb_body)
    return f(text3, table, pos)


def kernel(text, token_table, pos_embedding):
    text3 = text.astype(jnp.int32).reshape(_B, _NCHUNK, _CHUNK)
    return _emb(text3, token_table, pos_embedding)


# speedup vs baseline: 1.1922x; 1.1922x over previous
import functools

import jax
import jax.numpy as jnp
from jax import lax
from jax.experimental import pallas as pl
from jax.experimental.pallas import tpu as pltpu
from jax.experimental.pallas import tpu_sc as plsc

_VOCAB = 1000000
_D = 64
_B = 1024
_S = 200
_NC = 2
_NS = 16
_NW = _NC * _NS
_ROWS_PER_W = _B // _NW
_CHUNK = 100
_NCHUNK = _S // _CHUNK
_LANES = 16
_G = 4
_NSTEP = _ROWS_PER_W // _G
_NBUF = 2


def _emb_body(text_hbm, table_hbm, pos_hbm, out_hbm,
              idx_all, rows_buf, pos_v, gsem0, gsem1, ssem0, ssem1):
    wid = lax.axis_index("s") * _NC + lax.axis_index("c")
    row0 = wid * _ROWS_PER_W
    gsem = (gsem0, gsem1)
    ssem = (ssem0, ssem1)

    pltpu.sync_copy(text_hbm.at[pl.ds(row0, _ROWS_PER_W)], idx_all)
    pltpu.sync_copy(pos_hbm, pos_v)

    def issue_gathers(step, buf):
        for j in range(_G):
            for c in range(_NCHUNK):
                pltpu.async_copy(
                    table_hbm.at[idx_all.at[step * _G + j, c]],
                    rows_buf.at[buf, j, pl.ds(c * _CHUNK, _CHUNK)],
                    gsem[buf],
                )

    def wait_gathers(buf):
        for j in range(_G):
            pltpu.make_async_copy(
                table_hbm.at[pl.ds(0, _S)], rows_buf.at[buf, j], gsem[buf]
            ).wait()

    def issue_store(step, buf):
        pltpu.async_copy(
            rows_buf.at[buf],
            out_hbm.at[pl.ds(row0 + step * _G, _G)],
            ssem[buf],
        )

    def wait_store(buf):
        pltpu.make_async_copy(
            rows_buf.at[buf], out_hbm.at[pl.ds(0, _G)], ssem[buf]
        ).wait()

    def add_pos(buf):
        def body_r(r, carry):
            for q in range(_D // _LANES):
                cols = pl.ds(q * _LANES, _LANES)
                p = pos_v[r, cols]
                for j in range(_G):
                    sl = (buf, j, r, cols)
                    rows_buf[sl] = rows_buf[sl] + p
            return carry

        lax.fori_loop(0, _S, body_r, 0)

    def slot(b, step):
        nb = 1 - b

        @pl.when(step + 1 < _NSTEP)
        def _():
            @pl.when(step >= 1)
            def _():
                wait_store(nb)

            issue_gathers(step + 1, nb)

        wait_gathers(b)
        add_pos(b)
        issue_store(step, b)

    issue_gathers(0, 0)

    def body(gi, carry):
        slot(0, gi * _NBUF)
        slot(1, gi * _NBUF + 1)
        return carry

    lax.fori_loop(0, _NSTEP // _NBUF, body, 0)
    wait_store(0)
    wait_store(1)


@jax.jit
def _emb(text3, table, pos):
    mesh = plsc.VectorSubcoreMesh(core_axis_name="c", subcore_axis_name="s")
    f = functools.partial(
        pl.kernel,
        mesh=mesh,
        out_type=jax.ShapeDtypeStruct((_B, _S, _D), jnp.float32),
        scratch_types=[
            pltpu.VMEM((_ROWS_PER_W, _NCHUNK, _CHUNK), jnp.int32),
            pltpu.VMEM((_NBUF, _G, _S, _D), jnp.float32),
            pltpu.VMEM((_S, _D), jnp.float32),
            pltpu.SemaphoreType.DMA,
            pltpu.SemaphoreType.DMA,
            pltpu.SemaphoreType.DMA,
            pltpu.SemaphoreType.DMA,
        ],
        compiler_params=pltpu.CompilerParams(use_tc_tiling_on_sc=False),
    )(_emb_body)
    return f(text3, table, pos)


def kernel(text, token_table, pos_embedding):
    text3 = text.astype(jnp.int32).reshape(_B, _NCHUNK, _CHUNK)
    return _emb(text3, token_table, pos_embedding)
